# baseline (speedup 1.0000x reference)
import functools

import jax
import jax.numpy as jnp
from jax import lax
from jax.experimental import pallas as pl
from jax.experimental.pallas import tpu as pltpu
from jax.experimental.pallas import tpu_sc as plsc

TILE = 128


@functools.cache
def _make_row_gather(B, D):
    info = plsc.get_sparse_core_info()
    NC, NS = info.num_cores, info.num_subcores
    NW = NC * NS
    assert B % (8 * NW) == 0
    b_per_w = B // NW
    mesh = plsc.VectorSubcoreMesh(core_axis_name="c", subcore_axis_name="s")

    @functools.partial(
        pl.kernel,
        mesh=mesh,
        out_type=jax.ShapeDtypeStruct((B, D), jnp.float32),
        scratch_types=[
            pltpu.VMEM((b_per_w,), jnp.int32),
            pltpu.VMEM((b_per_w, D), jnp.float32),
            pltpu.SemaphoreType.DMA,
        ],
    )
    def gather(table_hbm, idx_hbm, out_hbm, idx_v, rows_v, sem):
        wid = lax.axis_index("s") * NC + lax.axis_index("c")
        base = wid * b_per_w
        pltpu.sync_copy(idx_hbm.at[pl.ds(base, b_per_w)], idx_v)
        pltpu.async_copy(table_hbm.at[idx_v], rows_v, sem).wait()
        pltpu.sync_copy(rows_v, out_hbm.at[pl.ds(base, b_per_w)])

    return gather


def _mm_body(gid_ref, x_ref, w_ref, b_ref, out_ref):
    del gid_ref
    out_ref[...] = (
        jnp.dot(x_ref[...], w_ref[0], preferred_element_type=jnp.float32)
        + b_ref[0]
    )


@functools.cache
def _make_grouped_mm(T, D):
    grid_spec = pltpu.PrefetchScalarGridSpec(
        num_scalar_prefetch=1,
        grid=(T,),
        in_specs=[
            pl.BlockSpec((TILE, D), lambda i, gid: (i, 0)),
            pl.BlockSpec((1, D, D), lambda i, gid: (gid[i], 0, 0)),
            pl.BlockSpec((1, 1, D), lambda i, gid: (gid[i], 0, 0)),
        ],
        out_specs=pl.BlockSpec((TILE, D), lambda i, gid: (i, 0)),
    )
    return pl.pallas_call(
        _mm_body,
        grid_spec=grid_spec,
        out_shape=jax.ShapeDtypeStruct((T * TILE, D), jnp.float32),
    )


def _routing_indices(actions, E, N, T):
    counts = jnp.bincount(actions, length=E).astype(jnp.int32)
    order = jnp.argsort(actions, stable=True).astype(jnp.int32)
    zero = jnp.zeros((1,), jnp.int32)
    start = jnp.concatenate([zero, jnp.cumsum(counts)]).astype(jnp.int32)
    padded = ((counts + TILE - 1) // TILE) * TILE
    pstart = jnp.concatenate([zero, jnp.cumsum(padded)]).astype(jnp.int32)
    B = T * TILE
    tile_starts = jnp.arange(T, dtype=jnp.int32) * TILE
    tile_gid = jnp.clip(
        jnp.searchsorted(pstart[1:], tile_starts, side="right"), 0, E - 1
    ).astype(jnp.int32)
    slots = jnp.arange(B, dtype=jnp.int32)
    se = tile_gid[slots // TILE]
    r = slots - pstart[se]
    valid = (r >= 0) & (r < counts[se])
    src = jnp.where(valid, order[jnp.clip(start[se] + r, 0, N - 1)], 0).astype(jnp.int32)
    j = jnp.arange(N, dtype=jnp.int32)
    sorted_e = actions[order]
    pslot = (pstart[sorted_e] + (j - start[sorted_e])).astype(jnp.int32)
    inv = jnp.zeros((N,), jnp.int32).at[order].set(pslot)
    return tile_gid, src, inv


def kernel(xs, mxs, actions, W, b):
    N, D = xs.shape
    E = W.shape[0]
    T = N // TILE + E
    tile_gid, src, inv = _routing_indices(actions, E, N, T)
    xs_sorted = _make_row_gather(T * TILE, D)(xs, src)
    ys_pad = _make_grouped_mm(T, D)(tile_gid, xs_sorted, W, b.reshape(E, 1, D))
    ys = _make_row_gather(N, D)(ys_pad, inv)
    return (ys, mxs, actions)


# speedup vs baseline: 1.3233x; 1.3233x over previous
import functools

import jax
import jax.numpy as jnp
from jax import lax
from jax.experimental import pallas as pl
from jax.experimental.pallas import tpu as pltpu
from jax.experimental.pallas import tpu_sc as plsc

TILE = 128


@functools.cache
def _make_row_gather(B, D):
    info = plsc.get_sparse_core_info()
    NC, NS = info.num_cores, info.num_subcores
    NW = NC * NS
    assert B % (8 * NW) == 0
    b_per_w = B // NW
    mesh = plsc.VectorSubcoreMesh(core_axis_name="c", subcore_axis_name="s")

    @functools.partial(
        pl.kernel,
        mesh=mesh,
        out_type=jax.ShapeDtypeStruct((B, D), jnp.float32),
        scratch_types=[
            pltpu.VMEM((b_per_w,), jnp.int32),
            pltpu.VMEM((b_per_w, D), jnp.float32),
            pltpu.SemaphoreType.DMA,
        ],
    )
    def gather(table_hbm, idx_hbm, out_hbm, idx_v, rows_v, sem):
        wid = lax.axis_index("s") * NC + lax.axis_index("c")
        base = wid * b_per_w
        pltpu.sync_copy(idx_hbm.at[pl.ds(base, b_per_w)], idx_v)
        pltpu.async_copy(table_hbm.at[idx_v], rows_v, sem).wait()
        pltpu.sync_copy(rows_v, out_hbm.at[pl.ds(base, b_per_w)])

    return gather


def _mm_body(gid_ref, x_ref, w_ref, b_ref, out_ref):
    del gid_ref
    out_ref[...] = (
        jnp.dot(x_ref[...], w_ref[0], preferred_element_type=jnp.float32)
        + b_ref[0]
    )


@functools.cache
def _make_grouped_mm(T, D):
    grid_spec = pltpu.PrefetchScalarGridSpec(
        num_scalar_prefetch=1,
        grid=(T,),
        in_specs=[
            pl.BlockSpec((TILE, D), lambda i, gid: (i, 0)),
            pl.BlockSpec((1, D, D), lambda i, gid: (gid[i], 0, 0)),
            pl.BlockSpec((1, 1, D), lambda i, gid: (gid[i], 0, 0)),
        ],
        out_specs=pl.BlockSpec((TILE, D), lambda i, gid: (i, 0)),
    )
    return pl.pallas_call(
        _mm_body,
        grid_spec=grid_spec,
        out_shape=jax.ShapeDtypeStruct((T * TILE, D), jnp.float32),
    )


def _routing_indices(actions, E, N, T):
    counts = jnp.bincount(actions, length=E).astype(jnp.int32)
    order = jnp.argsort(actions, stable=True).astype(jnp.int32)
    zero = jnp.zeros((1,), jnp.int32)
    start = jnp.concatenate([zero, jnp.cumsum(counts)]).astype(jnp.int32)
    padded = ((counts + TILE - 1) // TILE) * TILE
    pstart = jnp.concatenate([zero, jnp.cumsum(padded)]).astype(jnp.int32)
    B = T * TILE
    tile_starts = jnp.arange(T, dtype=jnp.int32) * TILE
    tile_gid = jnp.clip(
        jnp.searchsorted(pstart[1:], tile_starts, side="right"), 0, E - 1
    ).astype(jnp.int32)
    slots = jnp.arange(B, dtype=jnp.int32)
    se = tile_gid[slots // TILE]
    r = slots - pstart[se]
    valid = (r >= 0) & (r < counts[se])
    src = jnp.where(
        valid, order[jnp.clip(start[se] + r, 0, N - 1)], slots % N
    ).astype(jnp.int32)
    j = jnp.arange(N, dtype=jnp.int32)
    sorted_e = actions[order]
    pslot = (pstart[sorted_e] + (j - start[sorted_e])).astype(jnp.int32)
    inv = jnp.zeros((N,), jnp.int32).at[order].set(pslot)
    return tile_gid, src, inv


def kernel(xs, mxs, actions, W, b):
    N, D = xs.shape
    E = W.shape[0]
    T = N // TILE + E
    tile_gid, src, inv = _routing_indices(actions, E, N, T)
    xs_sorted = _make_row_gather(T * TILE, D)(xs, src)
    ys_pad = _make_grouped_mm(T, D)(tile_gid, xs_sorted, W, b.reshape(E, 1, D))
    ys = _make_row_gather(N, D)(ys_pad, inv)
    return (ys, mxs, actions)


# speedup vs baseline: 2.2476x; 1.6984x over previous
import functools

import jax
import jax.numpy as jnp
from jax import lax
from jax.experimental import pallas as pl
from jax.experimental.pallas import tpu as pltpu
from jax.experimental.pallas import tpu_sc as plsc

TILE = 128


@functools.cache
def _sc_info():
    info = plsc.get_sparse_core_info()
    return info.num_cores, info.num_subcores


@functools.cache
def _make_row_scatter(n_rows, B, D):
    NC, NS = _sc_info()
    NW = NC * NS
    assert n_rows % (8 * NW) == 0
    r_per_w = n_rows // NW
    mesh = plsc.VectorSubcoreMesh(core_axis_name="c", subcore_axis_name="s")

    @functools.partial(
        pl.kernel,
        mesh=mesh,
        out_type=jax.ShapeDtypeStruct((B, D), jnp.float32),
        scratch_types=[
            pltpu.VMEM((r_per_w,), jnp.int32),
            pltpu.VMEM((r_per_w, D), jnp.float32),
            pltpu.SemaphoreType.DMA,
        ],
    )
    def scatter(rows_hbm, idx_hbm, out_hbm, idx_v, rows_v, sem):
        wid = lax.axis_index("s") * NC + lax.axis_index("c")
        base = wid * r_per_w
        pltpu.sync_copy(idx_hbm.at[pl.ds(base, r_per_w)], idx_v)
        pltpu.sync_copy(rows_hbm.at[pl.ds(base, r_per_w)], rows_v)
        pltpu.async_copy(rows_v, out_hbm.at[idx_v], sem).wait()

    return scatter


@functools.cache
def _make_row_gather(B, D):
    NC, NS = _sc_info()
    NW = NC * NS
    assert B % (8 * NW) == 0
    b_per_w = B // NW
    mesh = plsc.VectorSubcoreMesh(core_axis_name="c", subcore_axis_name="s")

    @functools.partial(
        pl.kernel,
        mesh=mesh,
        out_type=jax.ShapeDtypeStruct((B, D), jnp.float32),
        scratch_types=[
            pltpu.VMEM((b_per_w,), jnp.int32),
            pltpu.VMEM((b_per_w, D), jnp.float32),
            pltpu.SemaphoreType.DMA,
        ],
    )
    def gather(table_hbm, idx_hbm, out_hbm, idx_v, rows_v, sem):
        wid = lax.axis_index("s") * NC + lax.axis_index("c")
        base = wid * b_per_w
        pltpu.sync_copy(idx_hbm.at[pl.ds(base, b_per_w)], idx_v)
        pltpu.async_copy(table_hbm.at[idx_v], rows_v, sem).wait()
        pltpu.sync_copy(rows_v, out_hbm.at[pl.ds(base, b_per_w)])

    return gather


def _mm_body(gid_ref, x_ref, w_ref, b_ref, out_ref):
    del gid_ref
    out_ref[...] = (
        jnp.dot(x_ref[...], w_ref[0], preferred_element_type=jnp.float32)
        + b_ref[0]
    )


@functools.cache
def _make_grouped_mm(T, D):
    grid_spec = pltpu.PrefetchScalarGridSpec(
        num_scalar_prefetch=1,
        grid=(T,),
        in_specs=[
            pl.BlockSpec((TILE, D), lambda i, gid: (i, 0)),
            pl.BlockSpec((1, D, D), lambda i, gid: (gid[i], 0, 0)),
            pl.BlockSpec((1, 1, D), lambda i, gid: (gid[i], 0, 0)),
        ],
        out_specs=pl.BlockSpec((TILE, D), lambda i, gid: (i, 0)),
    )
    return pl.pallas_call(
        _mm_body,
        grid_spec=grid_spec,
        out_shape=jax.ShapeDtypeStruct((T * TILE, D), jnp.float32),
    )


def _routing_indices(actions, E, T):
    onehot = (actions[:, None] == jnp.arange(E, dtype=jnp.int32)[None, :])
    csum = jnp.cumsum(onehot.astype(jnp.int32), axis=0)
    counts = csum[-1]
    rank = jnp.take_along_axis(csum, actions[:, None], axis=1)[:, 0] - 1
    padded = ((counts + TILE - 1) // TILE) * TILE
    pstart = jnp.concatenate(
        [jnp.zeros((1,), jnp.int32), jnp.cumsum(padded)]
    ).astype(jnp.int32)
    pslot = (pstart[actions] + rank).astype(jnp.int32)
    tile_starts = jnp.arange(T, dtype=jnp.int32) * TILE
    tile_gid = jnp.clip(
        jnp.sum((tile_starts[:, None] >= pstart[None, 1:]).astype(jnp.int32), axis=1),
        0, E - 1,
    ).astype(jnp.int32)
    return tile_gid, pslot


def kernel(xs, mxs, actions, W, b):
    N, D = xs.shape
    E = W.shape[0]
    T = N // TILE + E
    tile_gid, pslot = _routing_indices(actions, E, T)
    xs_sorted = _make_row_scatter(N, T * TILE, D)(xs, pslot)
    ys_pad = _make_grouped_mm(T, D)(tile_gid, xs_sorted, W, b.reshape(E, 1, D))
    ys = _make_row_gather(N, D)(ys_pad, pslot)
    return (ys, mxs, actions)


# speedup vs baseline: 2.5221x; 1.1221x over previous
import functools

import jax
import jax.numpy as jnp
from jax import lax
from jax.experimental import pallas as pl
from jax.experimental.pallas import tpu as pltpu
from jax.experimental.pallas import tpu_sc as plsc

TILE = 128


@functools.cache
def _sc_info():
    info = plsc.get_sparse_core_info()
    return info.num_cores, info.num_subcores


@functools.cache
def _make_row_scatter(n_rows, B, D):
    NC, NS = _sc_info()
    NW = NC * NS
    assert n_rows % (8 * NW) == 0
    r_per_w = n_rows // NW
    mesh = plsc.VectorSubcoreMesh(core_axis_name="c", subcore_axis_name="s")

    @functools.partial(
        pl.kernel,
        mesh=mesh,
        out_type=jax.ShapeDtypeStruct((B, D), jnp.float32),
        scratch_types=[
            pltpu.VMEM((r_per_w,), jnp.int32),
            pltpu.VMEM((r_per_w, D), jnp.float32),
            pltpu.SemaphoreType.DMA,
        ],
    )
    def scatter(rows_hbm, idx_hbm, out_hbm, idx_v, rows_v, sem):
        wid = lax.axis_index("s") * NC + lax.axis_index("c")
        base = wid * r_per_w
        pltpu.sync_copy(idx_hbm.at[pl.ds(base, r_per_w)], idx_v)
        pltpu.sync_copy(rows_hbm.at[pl.ds(base, r_per_w)], rows_v)
        pltpu.async_copy(rows_v, out_hbm.at[idx_v], sem).wait()

    return scatter


@functools.cache
def _make_row_gather(B, D):
    NC, NS = _sc_info()
    NW = NC * NS
    assert B % (8 * NW) == 0
    b_per_w = B // NW
    mesh = plsc.VectorSubcoreMesh(core_axis_name="c", subcore_axis_name="s")

    @functools.partial(
        pl.kernel,
        mesh=mesh,
        out_type=jax.ShapeDtypeStruct((B, D), jnp.float32),
        scratch_types=[
            pltpu.VMEM((b_per_w,), jnp.int32),
            pltpu.VMEM((b_per_w, D), jnp.float32),
            pltpu.SemaphoreType.DMA,
        ],
    )
    def gather(table_hbm, idx_hbm, out_hbm, idx_v, rows_v, sem):
        wid = lax.axis_index("s") * NC + lax.axis_index("c")
        base = wid * b_per_w
        pltpu.sync_copy(idx_hbm.at[pl.ds(base, b_per_w)], idx_v)
        pltpu.async_copy(table_hbm.at[idx_v], rows_v, sem).wait()
        pltpu.sync_copy(rows_v, out_hbm.at[pl.ds(base, b_per_w)])

    return gather


def _mm_body(gid_ref, valid_ref, x_ref, w_ref, b_ref, out_ref):
    del gid_ref

    @pl.when(valid_ref[pl.program_id(0)] != 0)
    def _():
        out_ref[...] = (
            jnp.dot(x_ref[...], w_ref[0], preferred_element_type=jnp.float32)
            + b_ref[0]
        )


@functools.cache
def _make_grouped_mm(T, D):
    grid_spec = pltpu.PrefetchScalarGridSpec(
        num_scalar_prefetch=2,
        grid=(T,),
        in_specs=[
            pl.BlockSpec((TILE, D), lambda i, gid, val: (i, 0)),
            pl.BlockSpec((1, D, D), lambda i, gid, val: (gid[i], 0, 0)),
            pl.BlockSpec((1, 1, D), lambda i, gid, val: (gid[i], 0, 0)),
        ],
        out_specs=pl.BlockSpec((TILE, D), lambda i, gid, val: (i, 0)),
    )
    return pl.pallas_call(
        _mm_body,
        grid_spec=grid_spec,
        out_shape=jax.ShapeDtypeStruct((T * TILE, D), jnp.float32),
    )


def _routing_indices(actions, E, T):
    onehot = (
        actions[:, None] == jnp.arange(E, dtype=jnp.int32)[None, :]
    ).astype(jnp.int32)
    csum = jnp.cumsum(onehot, axis=0)
    counts = csum[-1]
    padded = ((counts + TILE - 1) // TILE) * TILE
    pstart = jnp.concatenate(
        [jnp.zeros((1,), jnp.int32), jnp.cumsum(padded)]
    ).astype(jnp.int32)
    pslot = (
        jnp.sum(onehot * (csum + pstart[None, :E]), axis=1) - 1
    ).astype(jnp.int32)
    tile_starts = jnp.arange(T, dtype=jnp.int32) * TILE
    tile_gid = jnp.clip(
        jnp.sum((tile_starts[:, None] >= pstart[None, 1:]).astype(jnp.int32), axis=1),
        0, E - 1,
    ).astype(jnp.int32)
    tile_valid = (tile_starts < pstart[E]).astype(jnp.int32)
    return tile_gid, tile_valid, pslot


def kernel(xs, mxs, actions, W, b):
    N, D = xs.shape
    E = W.shape[0]
    T = N // TILE + E
    tile_gid, tile_valid, pslot = _routing_indices(actions, E, T)
    xs_sorted = _make_row_scatter(N, T * TILE, D)(xs, pslot)
    ys_pad = _make_grouped_mm(T, D)(
        tile_gid, tile_valid, xs_sorted, W, b.reshape(E, 1, D)
    )
    ys = _make_row_gather(N, D)(ys_pad, pslot)
    return (ys, mxs, actions)
